# Initial kernel scaffold; baseline (speedup 1.0000x reference)
import jax
import jax.numpy as jnp
from jax.experimental import pallas as pl


def kernel(x, edge_index, W0, att_src0, att_dst0, b0, W1, att_src1, att_dst1, b1):
    raise NotImplementedError("write your pallas kernel here")


# speedup vs baseline: 15.2482x; 15.2482x over previous
import functools

import jax
import jax.numpy as jnp
from jax import lax
from jax.experimental import pallas as pl
from jax.experimental.pallas import tpu as pltpu
from jax.experimental.pallas import tpu_sc as plsc

_N = 10000
_E = 320000
_F = 64
_HW = 128
_CW = 80
_WL = 80
_C = 80
_NC = 2
_NS = 16
_EPT = _E // (_NC * _NS)
_CHUNKS = _EPT // _C
_WR = 624
_WREM = _N - _WR * _NS
_ZR = 104
_RB = 1000
_GRID = _N // _RB
_HPAD = 8


def _proj_body(x_ref, w_ref, hxm_ref, scm_ref, dcm_ref, u_ref,
               hxt_ref, asct_ref, adct_ref):
    h = jnp.dot(x_ref[...], w_ref[...], preferred_element_type=jnp.float32)
    hxt_ref[...] = (jnp.dot(h, hxm_ref[...],
                            preferred_element_type=jnp.float32) + u_ref[...])
    asct_ref[...] = jnp.dot(h, scm_ref[...],
                            preferred_element_type=jnp.float32)
    adct_ref[...] = jnp.dot(h, dcm_ref[...],
                            preferred_element_type=jnp.float32)


def _proj(x, w, hxm, scm, dcm, u):
    din = x.shape[1]
    tab = jax.ShapeDtypeStruct((_N, _HW), jnp.float32)
    return pl.pallas_call(
        _proj_body,
        grid=(_GRID,),
        in_specs=[
            pl.BlockSpec((_RB, din), lambda i: (i, 0)),
            pl.BlockSpec((din, _F), lambda i: (0, 0)),
            pl.BlockSpec((_F, _HW), lambda i: (0, 0)),
            pl.BlockSpec((_F, _HW), lambda i: (0, 0)),
            pl.BlockSpec((_F, _HW), lambda i: (0, 0)),
            pl.BlockSpec((1, _HW), lambda i: (0, 0)),
        ],
        out_specs=[
            pl.BlockSpec((_RB, _HW), lambda i: (i, 0)),
            pl.BlockSpec((_RB, _HW), lambda i: (i, 0)),
            pl.BlockSpec((_RB, _HW), lambda i: (i, 0)),
        ],
        out_shape=[tab, tab, tab],
    )(x, w, hxm, scm, dcm, u)


def _mid_body(p0_ref, p1_ref, pm_ref, b0_ref, w1_ref, hxm_ref, scm_ref,
              dcm_ref, u_ref, hxt_ref, asct_ref, adct_ref):
    s = p0_ref[...] + p1_ref[...]
    den = jnp.dot(s, pm_ref[...], preferred_element_type=jnp.float32)
    h0 = s[:, :_F] / (den + 1e-16) + b0_ref[...]
    x1 = jnp.where(h0 > 0.0, h0, jnp.exp(h0) - 1.0)
    h1 = jnp.dot(x1, w1_ref[...], preferred_element_type=jnp.float32)
    hxt_ref[...] = (jnp.dot(h1, hxm_ref[...],
                            preferred_element_type=jnp.float32) + u_ref[...])
    asct_ref[...] = jnp.dot(h1, scm_ref[...],
                            preferred_element_type=jnp.float32)
    adct_ref[...] = jnp.dot(h1, dcm_ref[...],
                            preferred_element_type=jnp.float32)


def _mid(p0, p1, pm, b0, w1, hxm, scm, dcm, u):
    tab = jax.ShapeDtypeStruct((_N, _HW), jnp.float32)
    return pl.pallas_call(
        _mid_body,
        grid=(_GRID,),
        in_specs=[
            pl.BlockSpec((_RB, _CW), lambda i: (i, 0)),
            pl.BlockSpec((_RB, _CW), lambda i: (i, 0)),
            pl.BlockSpec((_CW, _F), lambda i: (0, 0)),
            pl.BlockSpec((1, _F), lambda i: (0, 0)),
            pl.BlockSpec((_F, _F), lambda i: (0, 0)),
            pl.BlockSpec((_F, _HW), lambda i: (0, 0)),
            pl.BlockSpec((_F, _HW), lambda i: (0, 0)),
            pl.BlockSpec((_F, _HW), lambda i: (0, 0)),
            pl.BlockSpec((1, _HW), lambda i: (0, 0)),
        ],
        out_specs=[
            pl.BlockSpec((_RB, _HW), lambda i: (i, 0)),
            pl.BlockSpec((_RB, _HW), lambda i: (i, 0)),
            pl.BlockSpec((_RB, _HW), lambda i: (i, 0)),
        ],
        out_shape=[tab, tab, tab],
    )(p0, p1, pm, b0, w1, hxm, scm, dcm, u)


def _fin_body(q0_ref, q1_ref, b1_ref, out_ref):
    s = q0_ref[...] + q1_ref[...]
    out_ref[...] = s[:, :_F] / (s[:, _F:_F + 1] + 1e-16) + b1_ref[...]


def _fin(q0, q1, b1):
    return pl.pallas_call(
        _fin_body,
        grid=(_GRID,),
        in_specs=[
            pl.BlockSpec((_RB, _CW), lambda i: (i, 0)),
            pl.BlockSpec((_RB, _CW), lambda i: (i, 0)),
            pl.BlockSpec((1, _F), lambda i: (0, 0)),
        ],
        out_specs=pl.BlockSpec((_RB, _F), lambda i: (i, 0)),
        out_shape=jax.ShapeDtypeStruct((_N, _F), jnp.float32),
    )(q0, q1, b1)


@functools.cache
def _sc_edge_w_fn():
    mesh = plsc.VectorSubcoreMesh(core_axis_name="c", subcore_axis_name="s")

    @functools.partial(
        pl.kernel,
        out_type=jax.ShapeDtypeStruct((_E * _WL,), jnp.float32),
        mesh=mesh,
        scratch_types=[
            pltpu.VMEM((_C,), jnp.int32),
            pltpu.VMEM((_C,), jnp.int32),
            pltpu.VMEM((_C, _HW), jnp.float32),
            pltpu.VMEM((_C, _HW), jnp.float32),
            pltpu.VMEM((_C * _WL,), jnp.float32),
            pltpu.SemaphoreType.DMA,
            pltpu.SemaphoreType.DMA,
        ],
    )
    def sc_edge_w(asct, adct, src, dst, wout,
                  sidx, didx, rs, rd, wbuf, sem_s, sem_d):
        cid = lax.axis_index("c")
        sid = lax.axis_index("s")
        ebase = (cid * _NS + sid) * _EPT

        @pl.loop(0, _CHUNKS)
        def chunk(k):
            off = ebase + k * _C
            pltpu.sync_copy(src.at[pl.ds(off, _C)], sidx)
            pltpu.sync_copy(dst.at[pl.ds(off, _C)], didx)
            cps = pltpu.async_copy(asct.at[sidx], rs, sem_s)
            cpd = pltpu.async_copy(adct.at[didx], rd, sem_d)
            cps.wait()
            cpd.wait()

            def erow(r, _):
                for c in range(_WL // 16):
                    sl = pl.ds(c * 16, 16)
                    a = rs[r, sl] + rd[r, sl]
                    a = jnp.where(a > 0.0, a, 0.2 * a)
                    wbuf[pl.ds(r * _WL + c * 16, 16)] = jnp.exp(a)
                return 0
            lax.fori_loop(0, _C, erow, 0)

            pltpu.sync_copy(wbuf, wout.at[pl.ds(off * _WL, _C * _WL)])

    return sc_edge_w


@functools.cache
def _sc_edge_acc_fn(out_ch):
    mesh = plsc.VectorSubcoreMesh(core_axis_name="c", subcore_axis_name="s")
    heads = _F // out_ch

    @functools.partial(
        pl.kernel,
        out_type=jax.ShapeDtypeStruct((_NC, _N, _CW), jnp.float32),
        mesh=mesh,
        scratch_types=[
            pltpu.VMEM((_C,), jnp.int32),
            pltpu.VMEM((_C,), jnp.int32),
            pltpu.VMEM((_C * _WL,), jnp.float32),
            pltpu.VMEM((_C, _HW), jnp.float32),
            pltpu.VMEM((_C, _CW), jnp.float32),
            pltpu.VMEM((_ZR, _CW), jnp.float32),
            pltpu.VMEM_SHARED((_N, _CW), jnp.float32),
            pltpu.SemaphoreType.DMA,
        ],
    )
    def sc_edge_acc(hxt, wsrc, src, dst, out,
                    sidx, didx, wb, rt, msg, zb, acc, sem):
        cid = lax.axis_index("c")
        sid = lax.axis_index("s")

        def zrow(i, _):
            for c in range(_CW // 16):
                zb[i, pl.ds(c * 16, 16)] = jnp.zeros((16,), jnp.float32)
            return 0
        lax.fori_loop(0, _ZR, zrow, 0)

        def zcopy(k, _):
            pltpu.sync_copy(zb, acc.at[pl.ds(sid * _WR + k * _ZR, _ZR)])
            return 0
        lax.fori_loop(0, _WR // _ZR, zcopy, 0)

        @pl.when(sid == _NS - 1)
        def _():
            pltpu.sync_copy(zb.at[pl.ds(0, _WREM)],
                            acc.at[pl.ds(_NS * _WR, _WREM)])

        plsc.subcore_barrier()

        ebase = (cid * _NS + sid) * _EPT

        @pl.loop(0, _CHUNKS)
        def chunk(k):
            off = ebase + k * _C
            pltpu.sync_copy(src.at[pl.ds(off, _C)], sidx)
            pltpu.sync_copy(dst.at[pl.ds(off, _C)], didx)
            pltpu.sync_copy(wsrc.at[pl.ds(off * _WL, _C * _WL)], wb)
            pltpu.async_copy(hxt.at[sidx], rt, sem).wait()

            def erow(r, _):
                for c in range(_CW // 16):
                    sl = pl.ds(c * 16, 16)
                    msg[r, sl] = wb[pl.ds(r * _WL + c * 16, 16)] * rt[r, sl]
                return 0
            lax.fori_loop(0, _C, erow, 0)

            pltpu.sync_copy(msg, acc.at[didx], add=True)

        plsc.subcore_barrier()
        row0 = sid * _WR
        pltpu.sync_copy(acc.at[pl.ds(row0, _WR)],
                        out.at[cid, pl.ds(row0, _WR)])

        @pl.when(sid == _NS - 1)
        def _():
            pltpu.sync_copy(acc.at[pl.ds(_NS * _WR, _WREM)],
                            out.at[cid, pl.ds(_NS * _WR, _WREM)])

    return sc_edge_acc


def _selector_mats(att_src, att_dst, heads):
    out_ch = _F // heads
    att_s = att_src.reshape(heads, out_ch).astype(jnp.float32)
    att_d = att_dst.reshape(heads, out_ch).astype(jnp.float32)
    eyeh = jnp.eye(heads, dtype=jnp.float32)
    a_s = (att_s[:, :, None] * eyeh[:, None, :]).reshape(_F, heads)
    a_d = (att_d[:, :, None] * eyeh[:, None, :]).reshape(_F, heads)
    cols = jnp.arange(_HW)
    rows = jnp.arange(heads)[:, None]
    m = (((cols[None, :] < _F) & ((cols[None, :] // out_ch) == rows))
         | (cols[None, :] == _F + rows)).astype(jnp.float32)
    scm = a_s @ m
    dcm = a_d @ m
    hxm = jnp.concatenate(
        [jnp.eye(_F, dtype=jnp.float32),
         jnp.zeros((_F, _HW - _F), jnp.float32)], axis=1)
    u = (((cols >= _F) & (cols < _F + heads)).astype(jnp.float32))[None, :]
    return hxm, scm, dcm, u


def kernel(x, edge_index, W0, att_src0, att_dst0, b0, W1, att_src1,
           att_dst1, b1):
    src = edge_index[0]
    dst = edge_index[1]

    hxm0, scm0, dcm0, u0 = _selector_mats(att_src0, att_dst0, 8)
    hxt0, asct0, adct0 = _proj(x, W0, hxm0, scm0, dcm0, u0)

    w0 = _sc_edge_w_fn()(asct0, adct0, src, dst)
    part0 = _sc_edge_acc_fn(8)(hxt0, w0, src, dst)

    pm = ((jnp.arange(_CW)[:, None] - _F)
          == (jnp.arange(_F)[None, :] // _HPAD)).astype(jnp.float32)
    hxm1, scm1, dcm1, u1 = _selector_mats(att_src1, att_dst1, 1)
    hxt1, asct1, adct1 = _mid(part0[0], part0[1], pm, b0[None, :], W1,
                              hxm1, scm1, dcm1, u1)

    w1 = _sc_edge_w_fn()(asct1, adct1, src, dst)
    part1 = _sc_edge_acc_fn(64)(hxt1, w1, src, dst)

    return _fin(part1[0], part1[1], b1[None, :])


# speedup vs baseline: 34.3068x; 2.2499x over previous
import functools

import jax
import jax.numpy as jnp
from jax import lax
from jax.experimental import pallas as pl
from jax.experimental.pallas import tpu as pltpu
from jax.experimental.pallas import tpu_sc as plsc

_N = 10000
_E = 320000
_F = 64
_HW = 128
_CW = 80
_WL = 80
_C = 80
_NC = 2
_NS = 16
_EPT = _E // (_NC * _NS)
_CHUNKS = _EPT // _C
_WR = 624
_WREM = _N - _WR * _NS
_ZR = 104
_RB = 1000
_GRID = _N // _RB
_HPAD = 8


def _proj_body(x_ref, w_ref, hxm_ref, scm_ref, dcm_ref, u_ref,
               hxt_ref, asct_ref, adct_ref):
    h = jnp.dot(x_ref[...], w_ref[...], preferred_element_type=jnp.float32)
    hxt_ref[...] = (jnp.dot(h, hxm_ref[...],
                            preferred_element_type=jnp.float32) + u_ref[...])
    asct_ref[...] = jnp.dot(h, scm_ref[...],
                            preferred_element_type=jnp.float32)
    adct_ref[...] = jnp.dot(h, dcm_ref[...],
                            preferred_element_type=jnp.float32)


def _proj(x, w, hxm, scm, dcm, u):
    din = x.shape[1]
    tab = jax.ShapeDtypeStruct((_N, _HW), jnp.float32)
    return pl.pallas_call(
        _proj_body,
        grid=(_GRID,),
        in_specs=[
            pl.BlockSpec((_RB, din), lambda i: (i, 0)),
            pl.BlockSpec((din, _F), lambda i: (0, 0)),
            pl.BlockSpec((_F, _HW), lambda i: (0, 0)),
            pl.BlockSpec((_F, _HW), lambda i: (0, 0)),
            pl.BlockSpec((_F, _HW), lambda i: (0, 0)),
            pl.BlockSpec((1, _HW), lambda i: (0, 0)),
        ],
        out_specs=[
            pl.BlockSpec((_RB, _HW), lambda i: (i, 0)),
            pl.BlockSpec((_RB, _HW), lambda i: (i, 0)),
            pl.BlockSpec((_RB, _HW), lambda i: (i, 0)),
        ],
        out_shape=[tab, tab, tab],
    )(x, w, hxm, scm, dcm, u)


def _mid_body(p0_ref, p1_ref, pm_ref, b0_ref, w1_ref, hxm_ref, scm_ref,
              dcm_ref, u_ref, hxt_ref, asct_ref, adct_ref):
    s = p0_ref[...] + p1_ref[...]
    den = jnp.dot(s, pm_ref[...], preferred_element_type=jnp.float32)
    h0 = s[:, :_F] / (den + 1e-16) + b0_ref[...]
    x1 = jnp.where(h0 > 0.0, h0, jnp.exp(h0) - 1.0)
    h1 = jnp.dot(x1, w1_ref[...], preferred_element_type=jnp.float32)
    hxt_ref[...] = (jnp.dot(h1, hxm_ref[...],
                            preferred_element_type=jnp.float32) + u_ref[...])
    asct_ref[...] = jnp.dot(h1, scm_ref[...],
                            preferred_element_type=jnp.float32)
    adct_ref[...] = jnp.dot(h1, dcm_ref[...],
                            preferred_element_type=jnp.float32)


def _mid(p0, p1, pm, b0, w1, hxm, scm, dcm, u):
    tab = jax.ShapeDtypeStruct((_N, _HW), jnp.float32)
    return pl.pallas_call(
        _mid_body,
        grid=(_GRID,),
        in_specs=[
            pl.BlockSpec((_RB, _CW), lambda i: (i, 0)),
            pl.BlockSpec((_RB, _CW), lambda i: (i, 0)),
            pl.BlockSpec((_CW, _F), lambda i: (0, 0)),
            pl.BlockSpec((1, _F), lambda i: (0, 0)),
            pl.BlockSpec((_F, _F), lambda i: (0, 0)),
            pl.BlockSpec((_F, _HW), lambda i: (0, 0)),
            pl.BlockSpec((_F, _HW), lambda i: (0, 0)),
            pl.BlockSpec((_F, _HW), lambda i: (0, 0)),
            pl.BlockSpec((1, _HW), lambda i: (0, 0)),
        ],
        out_specs=[
            pl.BlockSpec((_RB, _HW), lambda i: (i, 0)),
            pl.BlockSpec((_RB, _HW), lambda i: (i, 0)),
            pl.BlockSpec((_RB, _HW), lambda i: (i, 0)),
        ],
        out_shape=[tab, tab, tab],
    )(p0, p1, pm, b0, w1, hxm, scm, dcm, u)


def _fin_body(q0_ref, q1_ref, b1_ref, out_ref):
    s = q0_ref[...] + q1_ref[...]
    out_ref[...] = s[:, :_F] / (s[:, _F:_F + 1] + 1e-16) + b1_ref[...]


def _fin(q0, q1, b1):
    return pl.pallas_call(
        _fin_body,
        grid=(_GRID,),
        in_specs=[
            pl.BlockSpec((_RB, _CW), lambda i: (i, 0)),
            pl.BlockSpec((_RB, _CW), lambda i: (i, 0)),
            pl.BlockSpec((1, _F), lambda i: (0, 0)),
        ],
        out_specs=pl.BlockSpec((_RB, _F), lambda i: (i, 0)),
        out_shape=jax.ShapeDtypeStruct((_N, _F), jnp.float32),
    )(q0, q1, b1)


@functools.cache
def _sc_edge_w_fn():
    mesh = plsc.VectorSubcoreMesh(core_axis_name="c", subcore_axis_name="s")

    @functools.partial(
        pl.kernel,
        out_type=jax.ShapeDtypeStruct((_E, _WL), jnp.float32),
        mesh=mesh,
        scratch_types=[
            pltpu.VMEM((_C,), jnp.int32),
            pltpu.VMEM((_C,), jnp.int32),
            pltpu.VMEM((_C, _HW), jnp.float32),
            pltpu.VMEM((_C, _HW), jnp.float32),
            pltpu.VMEM((_C, _HW), jnp.float32),
            pltpu.VMEM((_C, _WL), jnp.float32),
            pltpu.SemaphoreType.DMA,
        ],
    )
    def sc_edge_w(asct, adct, hxt, src, dst, mout,
                  sidx, didx, rs, rd, rt, mbuf, sem):
        cid = lax.axis_index("c")
        sid = lax.axis_index("s")
        ebase = (cid * _NS + sid) * _EPT

        @pl.loop(0, _CHUNKS)
        def chunk(k):
            off = ebase + k * _C
            pltpu.sync_copy(src.at[pl.ds(off, _C)], sidx)
            cps = pltpu.async_copy(asct.at[sidx], rs, sem)
            cpt = pltpu.async_copy(hxt.at[sidx], rt, sem)
            pltpu.sync_copy(dst.at[pl.ds(off, _C)], didx)
            cpd = pltpu.async_copy(adct.at[didx], rd, sem)
            cps.wait()
            cpt.wait()
            cpd.wait()

            def erow(r, _):
                for c in range(_WL // 16):
                    sl = pl.ds(c * 16, 16)
                    a = rs[r, sl] + rd[r, sl]
                    a = jnp.where(a > 0.0, a, 0.2 * a)
                    mbuf[r, sl] = jnp.exp(a) * rt[r, sl]
                return 0
            lax.fori_loop(0, _C, erow, 0)

            pltpu.sync_copy(mbuf, mout.at[pl.ds(off, _C)])

    return sc_edge_w


@functools.cache
def _sc_edge_acc_fn():
    mesh = plsc.VectorSubcoreMesh(core_axis_name="c", subcore_axis_name="s")

    @functools.partial(
        pl.kernel,
        out_type=jax.ShapeDtypeStruct((_NC, _N, _CW), jnp.float32),
        mesh=mesh,
        scratch_types=[
            pltpu.VMEM((_C,), jnp.int32),
            pltpu.VMEM((_C, _WL), jnp.float32),
            pltpu.VMEM((_ZR, _CW), jnp.float32),
            pltpu.VMEM_SHARED((_N, _CW), jnp.float32),
        ],
    )
    def sc_edge_acc(msrc, dst, out, didx, msg, zb, acc):
        cid = lax.axis_index("c")
        sid = lax.axis_index("s")

        def zrow(i, _):
            for c in range(_CW // 16):
                zb[i, pl.ds(c * 16, 16)] = jnp.zeros((16,), jnp.float32)
            return 0
        lax.fori_loop(0, _ZR, zrow, 0)

        def zcopy(k, _):
            pltpu.sync_copy(zb, acc.at[pl.ds(sid * _WR + k * _ZR, _ZR)])
            return 0
        lax.fori_loop(0, _WR // _ZR, zcopy, 0)

        @pl.when(sid == _NS - 1)
        def _():
            pltpu.sync_copy(zb.at[pl.ds(0, _WREM)],
                            acc.at[pl.ds(_NS * _WR, _WREM)])

        plsc.subcore_barrier()

        ebase = (cid * _NS + sid) * _EPT

        @pl.loop(0, _CHUNKS)
        def chunk(k):
            off = ebase + k * _C
            pltpu.sync_copy(dst.at[pl.ds(off, _C)], didx)
            pltpu.sync_copy(msrc.at[pl.ds(off, _C)], msg)
            pltpu.sync_copy(msg, acc.at[didx], add=True)

        plsc.subcore_barrier()
        row0 = sid * _WR
        pltpu.sync_copy(acc.at[pl.ds(row0, _WR)],
                        out.at[cid, pl.ds(row0, _WR)])

        @pl.when(sid == _NS - 1)
        def _():
            pltpu.sync_copy(acc.at[pl.ds(_NS * _WR, _WREM)],
                            out.at[cid, pl.ds(_NS * _WR, _WREM)])

    return sc_edge_acc


def _selector_mats(att_src, att_dst, heads):
    out_ch = _F // heads
    att_s = att_src.reshape(heads, out_ch).astype(jnp.float32)
    att_d = att_dst.reshape(heads, out_ch).astype(jnp.float32)
    eyeh = jnp.eye(heads, dtype=jnp.float32)
    a_s = (att_s[:, :, None] * eyeh[:, None, :]).reshape(_F, heads)
    a_d = (att_d[:, :, None] * eyeh[:, None, :]).reshape(_F, heads)
    cols = jnp.arange(_HW)
    rows = jnp.arange(heads)[:, None]
    m = (((cols[None, :] < _F) & ((cols[None, :] // out_ch) == rows))
         | (cols[None, :] == _F + rows)).astype(jnp.float32)
    scm = a_s @ m
    dcm = a_d @ m
    hxm = jnp.concatenate(
        [jnp.eye(_F, dtype=jnp.float32),
         jnp.zeros((_F, _HW - _F), jnp.float32)], axis=1)
    u = (((cols >= _F) & (cols < _F + heads)).astype(jnp.float32))[None, :]
    return hxm, scm, dcm, u


def kernel(x, edge_index, W0, att_src0, att_dst0, b0, W1, att_src1,
           att_dst1, b1):
    src = edge_index[0]
    dst = edge_index[1]

    hxm0, scm0, dcm0, u0 = _selector_mats(att_src0, att_dst0, 8)
    hxt0, asct0, adct0 = _proj(x, W0, hxm0, scm0, dcm0, u0)

    m0 = _sc_edge_w_fn()(asct0, adct0, hxt0, src, dst)
    part0 = _sc_edge_acc_fn()(m0, dst)

    pm = ((jnp.arange(_CW)[:, None] - _F)
          == (jnp.arange(_F)[None, :] // _HPAD)).astype(jnp.float32)
    hxm1, scm1, dcm1, u1 = _selector_mats(att_src1, att_dst1, 1)
    hxt1, asct1, adct1 = _mid(part0[0], part0[1], pm, b0[None, :], W1,
                              hxm1, scm1, dcm1, u1)

    m1 = _sc_edge_w_fn()(asct1, adct1, hxt1, src, dst)
    part1 = _sc_edge_acc_fn()(m1, dst)

    return _fin(part1[0], part1[1], b1[None, :])
